# baseline (speedup 1.0000x reference)
import functools

import jax
import jax.numpy as jnp
from jax.experimental import pallas as pl
from jax.experimental.pallas import tpu as pltpu

N = 4096
D = 128
TEMP = 0.1
K = max(1, int(0.5 * (N - 1)))
BLOCK = 1024
N_ITERS = 16
LO0 = -1.1 / TEMP
HI0 = 1.1 / TEMP


def _normalize_kernel(q_ref, p_ref, qn_ref, pn_ref):
    q = q_ref[...]
    p = p_ref[...]
    qs = (1.0 / TEMP) / jnp.maximum(
        jnp.sqrt(jnp.sum(q * q, axis=1, keepdims=True)), 1e-12)
    ps = 1.0 / jnp.maximum(
        jnp.sqrt(jnp.sum(p * p, axis=1, keepdims=True)), 1e-12)
    qn_ref[...] = q * qs
    pn_ref[...] = p * ps


def _loss_block_kernel(q_ref, p_ref, pblk_ref, out_ref):
    qb = q_ref[...]
    pf = p_ref[...]

    logits = jax.lax.dot_general(
        qb, pf, dimension_numbers=(((1,), (1,)), ((), ())),
        preferred_element_type=jnp.float32,
    )

    pos = jnp.sum(qb * pblk_ref[...], axis=1, keepdims=True)

    kf = jnp.float32(K)

    def bs_body(_, carry):
        lo, hi = carry
        mid = 0.5 * (lo + hi)
        cnt = jnp.sum((logits < mid).astype(jnp.float32), axis=1, keepdims=True)
        cnt = cnt - (pos < mid).astype(jnp.float32)
        ge = cnt >= kf
        return jnp.where(ge, lo, mid), jnp.where(ge, mid, hi)

    lo = jnp.full((BLOCK, 1), LO0, jnp.float32)
    hi = jnp.full((BLOCK, 1), HI0, jnp.float32)
    lo, hi = jax.lax.fori_loop(0, N_ITERS, bs_body, (lo, hi))
    t = 0.5 * (lo + hi)

    m = jnp.maximum(pos, t)
    ex = jnp.exp(jnp.minimum(logits, t) - m)
    s = jnp.sum(ex, axis=1, keepdims=True)
    s = (s - jnp.exp(jnp.minimum(pos, t) - m)
         - (N - 1 - K) * jnp.exp(t - m) + jnp.exp(pos - m))
    losses = -pos + m + jnp.log(s)

    out_ref[...] = jnp.sum(losses).reshape(1, 1, 1)


@jax.jit
def kernel(query, positive_key):
    qn, pn = pl.pallas_call(
        _normalize_kernel,
        in_specs=[
            pl.BlockSpec((N, D), lambda: (0, 0)),
            pl.BlockSpec((N, D), lambda: (0, 0)),
        ],
        out_specs=[
            pl.BlockSpec((N, D), lambda: (0, 0)),
            pl.BlockSpec((N, D), lambda: (0, 0)),
        ],
        out_shape=[
            jax.ShapeDtypeStruct((N, D), jnp.float32),
            jax.ShapeDtypeStruct((N, D), jnp.float32),
        ],
    )(query, positive_key)

    out = pl.pallas_call(
        _loss_block_kernel,
        grid=(N // BLOCK,),
        in_specs=[
            pl.BlockSpec((BLOCK, D), lambda i: (i, 0)),
            pl.BlockSpec((N, D), lambda i: (0, 0)),
            pl.BlockSpec((BLOCK, D), lambda i: (i, 0)),
        ],
        out_specs=pl.BlockSpec((1, 1, 1), lambda i: (i, 0, 0)),
        out_shape=jax.ShapeDtypeStruct((N // BLOCK, 1, 1), jnp.float32),
        compiler_params=pltpu.CompilerParams(
            dimension_semantics=("parallel",),
        ),
    )(qn, pn, pn)
    return jnp.sum(out) / N


# speedup vs baseline: 1.2689x; 1.2689x over previous
import functools

import jax
import jax.numpy as jnp
from jax.experimental import pallas as pl
from jax.experimental.pallas import tpu as pltpu

N = 4096
D = 128
TEMP = 0.1
K = max(1, int(0.5 * (N - 1)))
BLOCK = 1024
N_ITERS = 12
LO0 = -1.1 / TEMP
HI0 = 1.1 / TEMP


def _normalize_kernel(q_ref, p_ref, qn_ref, pn_ref):
    q = q_ref[...]
    p = p_ref[...]
    qs = (1.0 / TEMP) / jnp.maximum(
        jnp.sqrt(jnp.sum(q * q, axis=1, keepdims=True)), 1e-12)
    ps = 1.0 / jnp.maximum(
        jnp.sqrt(jnp.sum(p * p, axis=1, keepdims=True)), 1e-12)
    qn_ref[...] = q * qs
    pn_ref[...] = p * ps


def _loss_block_kernel(q_ref, p_ref, pblk_ref, out_ref):
    qb = q_ref[...]
    pf = p_ref[...]

    logits = jax.lax.dot_general(
        qb, pf, dimension_numbers=(((1,), (1,)), ((), ())),
        preferred_element_type=jnp.float32,
    )

    pos = jnp.sum(qb * pblk_ref[...], axis=1, keepdims=True)

    kf = jnp.float32(K)

    def bs_body(_, carry):
        lo, hi = carry
        mid = 0.5 * (lo + hi)
        cnt = jnp.sum((logits < mid).astype(jnp.float32), axis=1, keepdims=True)
        cnt = cnt - (pos < mid).astype(jnp.float32)
        ge = cnt >= kf
        return jnp.where(ge, lo, mid), jnp.where(ge, mid, hi)

    lo = jnp.full((BLOCK, 1), LO0, jnp.float32)
    hi = jnp.full((BLOCK, 1), HI0, jnp.float32)
    lo, hi = jax.lax.fori_loop(0, N_ITERS, bs_body, (lo, hi))
    t = 0.5 * (lo + hi)

    ex = jnp.exp(jnp.minimum(logits, t))
    s = jnp.sum(ex, axis=1, keepdims=True)
    s = (s - jnp.exp(jnp.minimum(pos, t))
         - (N - 1 - K) * jnp.exp(t) + jnp.exp(pos))
    losses = -pos + jnp.log(s)

    out_ref[...] = jnp.sum(losses).reshape(1, 1, 1)


@jax.jit
def kernel(query, positive_key):
    qn, pn = pl.pallas_call(
        _normalize_kernel,
        in_specs=[
            pl.BlockSpec((N, D), lambda: (0, 0)),
            pl.BlockSpec((N, D), lambda: (0, 0)),
        ],
        out_specs=[
            pl.BlockSpec((N, D), lambda: (0, 0)),
            pl.BlockSpec((N, D), lambda: (0, 0)),
        ],
        out_shape=[
            jax.ShapeDtypeStruct((N, D), jnp.float32),
            jax.ShapeDtypeStruct((N, D), jnp.float32),
        ],
    )(query, positive_key)

    out = pl.pallas_call(
        _loss_block_kernel,
        grid=(N // BLOCK,),
        in_specs=[
            pl.BlockSpec((BLOCK, D), lambda i: (i, 0)),
            pl.BlockSpec((N, D), lambda i: (0, 0)),
            pl.BlockSpec((BLOCK, D), lambda i: (i, 0)),
        ],
        out_specs=pl.BlockSpec((1, 1, 1), lambda i: (i, 0, 0)),
        out_shape=jax.ShapeDtypeStruct((N // BLOCK, 1, 1), jnp.float32),
        compiler_params=pltpu.CompilerParams(
            dimension_semantics=("parallel",),
        ),
    )(qn, pn, pn)
    return jnp.sum(out) / N
